# baseline (speedup 1.0000x reference)
import functools

import jax
import jax.numpy as jnp
from jax import lax
from jax.experimental import pallas as pl
from jax.experimental.pallas import tpu as pltpu
from jax.experimental.pallas import tpu_sc as plsc

N = 10000
E = 320000
D = 128

NC = 2
NS = 16
NW = NC * NS
EP = E // NW
C = 80
NCH = EP // C
NB = 4
NG = (NCH - NB) // NB
TAIL = NCH - NB * (NG + 1)
NPAD = 10240
ROWS_PER_SUB = NPAD // NS
ZR = 8


def _segsum_sc(h, src, dst):
    mesh = plsc.VectorSubcoreMesh(core_axis_name="core", subcore_axis_name="subcore")

    @functools.partial(
        pl.kernel,
        out_type=[
            jax.ShapeDtypeStruct((NPAD, D), jnp.float32),
            jax.ShapeDtypeStruct((NPAD, D), jnp.float32),
        ],
        mesh=mesh,
        scratch_types=[
            pltpu.VMEM((C,), jnp.int32),
            pltpu.VMEM((C,), jnp.int32),
            pltpu.VMEM((C,), jnp.int32),
            pltpu.VMEM((C,), jnp.int32),
            pltpu.VMEM((C,), jnp.int32),
            pltpu.VMEM((C,), jnp.int32),
            pltpu.VMEM((C,), jnp.int32),
            pltpu.VMEM((C,), jnp.int32),
            pltpu.VMEM((C, D), jnp.float32),
            pltpu.VMEM((C, D), jnp.float32),
            pltpu.VMEM((C, D), jnp.float32),
            pltpu.VMEM((C, D), jnp.float32),
            pltpu.VMEM((ZR, D), jnp.float32),
            pltpu.VMEM_SHARED((NPAD, D), jnp.float32),
            pltpu.SemaphoreType.DMA,
            pltpu.SemaphoreType.DMA,
            pltpu.SemaphoreType.DMA,
            pltpu.SemaphoreType.DMA,
            pltpu.SemaphoreType.DMA,
            pltpu.SemaphoreType.DMA,
            pltpu.SemaphoreType.DMA,
            pltpu.SemaphoreType.DMA,
            pltpu.SemaphoreType.DMA,
            pltpu.SemaphoreType.DMA,
            pltpu.SemaphoreType.DMA,
            pltpu.SemaphoreType.DMA,
            pltpu.SemaphoreType.DMA,
            pltpu.SemaphoreType.DMA,
            pltpu.SemaphoreType.DMA,
            pltpu.SemaphoreType.DMA,
        ],
    )
    def seg_kernel(h_hbm, src_hbm, dst_hbm, o0_hbm, o1_hbm,
                   sv0, sv1, sv2, sv3, dv0, dv1, dv2, dv3,
                   r0, r1, r2, r3, zbuf, acc,
                   g0, g1, g2, g3, s0, s1, s2, s3,
                   ds0, ds1, ds2, ds3, is0, is1, is2, is3):
        srcv = [sv0, sv1, sv2, sv3]
        dstv = [dv0, dv1, dv2, dv3]
        rows = [r0, r1, r2, r3]
        gsem = [g0, g1, g2, g3]
        ssem = [s0, s1, s2, s3]
        dsem = [ds0, ds1, ds2, ds3]
        isem = [is0, is1, is2, is3]
        cid = lax.axis_index("core")
        sid = lax.axis_index("subcore")
        w = sid * NC + cid

        def start_src_idx(i, b):
            pltpu.async_copy(src_hbm.at[pl.ds(w * EP + i * C, C)], srcv[b],
                             isem[b])

        def start_dst_idx(i, b):
            pltpu.async_copy(dst_hbm.at[pl.ds(w * EP + i * C, C)], dstv[b],
                             dsem[b])

        def start_gather(i, b):
            pltpu.make_async_copy(src_hbm.at[pl.ds(w * EP + i * C, C)],
                                  srcv[b], isem[b]).wait()
            pltpu.async_copy(h_hbm.at[srcv[b]], rows[b], gsem[b])

        def finish_chunk(i, b):
            pltpu.make_async_copy(dst_hbm.at[pl.ds(w * EP + i * C, C)],
                                  dstv[b], dsem[b]).wait()
            pltpu.make_async_copy(h_hbm.at[srcv[b]], rows[b], gsem[b]).wait()
            pltpu.async_copy(rows[b], acc.at[dstv[b]], ssem[b], add=True)

        def wait_scatter(b):
            pltpu.make_async_copy(rows[b], acc.at[dstv[b]], ssem[b]).wait()

        for b in range(NB):
            start_src_idx(b, b)
            start_dst_idx(b, b)

        @pl.loop(0, ZR)
        def _(i):
            @pl.loop(0, D, step=16)
            def _(j):
                zbuf[i, pl.ds(j, 16)] = jnp.zeros((16,), jnp.float32)

        @pl.loop(0, ROWS_PER_SUB // ZR)
        def _(k):
            pltpu.sync_copy(zbuf, acc.at[pl.ds(sid * ROWS_PER_SUB + k * ZR, ZR)])

        plsc.subcore_barrier()

        for b in range(NB):
            start_gather(b, b)

        @pl.loop(0, NG)
        def _(g):
            i0 = g * NB
            for b in range(NB):
                finish_chunk(i0 + b, b)
                start_src_idx(i0 + NB + b, b)
            for b in range(NB):
                wait_scatter(b)
                start_dst_idx(i0 + NB + b, b)
                start_gather(i0 + NB + b, b)

        i0 = NG * NB
        for b in range(NB):
            finish_chunk(i0 + b, b)
            if b < TAIL:
                start_src_idx(i0 + NB + b, b)
        for t in range(TAIL):
            wait_scatter(t)
            start_dst_idx(i0 + NB + t, t)
            start_gather(i0 + NB + t, t)
        for t in range(TAIL):
            finish_chunk(i0 + NB + t, t)
        for b in range(NB):
            wait_scatter(b)

        plsc.subcore_barrier()

        row0 = sid * ROWS_PER_SUB

        @pl.when(cid == 0)
        def _():
            pltpu.sync_copy(acc.at[pl.ds(row0, ROWS_PER_SUB)],
                            o0_hbm.at[pl.ds(row0, ROWS_PER_SUB)])

        @pl.when(cid == 1)
        def _():
            pltpu.sync_copy(acc.at[pl.ds(row0, ROWS_PER_SUB)],
                            o1_hbm.at[pl.ds(row0, ROWS_PER_SUB)])

    return seg_kernel(h, src, dst)


ROW_BLK = 1000


def _mlp_body(final_relu, x_ref, p0_ref, p1_ref, wa_ref, ba_ref, wb_ref, bb_ref,
              o_ref):
    z = x_ref[...] + p0_ref[...] + p1_ref[...]
    t = jnp.dot(z, wa_ref[...], preferred_element_type=jnp.float32)
    t = jnp.maximum(t + ba_ref[...], 0.0)
    o = jnp.dot(t, wb_ref[...], preferred_element_type=jnp.float32)
    o = o + bb_ref[...]
    if final_relu:
        o = jnp.maximum(o, 0.0)
    o_ref[...] = o


def _mlp(x, p0, p1, Wa, ba, Wb, bb, final_relu):
    row_spec = pl.BlockSpec((ROW_BLK, D), lambda i: (i, 0))
    w_spec = pl.BlockSpec((D, D), lambda i: (0, 0))
    b_spec = pl.BlockSpec((1, D), lambda i: (0, 0))
    return pl.pallas_call(
        functools.partial(_mlp_body, final_relu),
        grid=(N // ROW_BLK,),
        in_specs=[row_spec, row_spec, row_spec, w_spec, b_spec, w_spec, b_spec],
        out_specs=row_spec,
        out_shape=jax.ShapeDtypeStruct((N, D), jnp.float32),
    )(x, p0, p1, Wa, ba.reshape(1, D), Wb, bb.reshape(1, D))


def kernel(x, edge_index, W1a, b1a, W1b, b1b, W2a, b2a, W2b, b2b):
    src = edge_index[0]
    dst = edge_index[1]
    p0, p1 = _segsum_sc(x, src, dst)
    h = _mlp(x, p0, p1, W1a, b1a, W1b, b1b, final_relu=True)
    q0, q1 = _segsum_sc(h, src, dst)
    return _mlp(h, q0, q1, W2a, b2a, W2b, b2b, final_relu=False)


# speedup vs baseline: 1.0148x; 1.0148x over previous
import functools

import jax
import jax.numpy as jnp
from jax import lax
from jax.experimental import pallas as pl
from jax.experimental.pallas import tpu as pltpu
from jax.experimental.pallas import tpu_sc as plsc

N = 10000
E = 320000
D = 128

NC = 2
NS = 16
NW = NC * NS
EP = E // NW
C = 80
NCH = EP // C
NB = 4
NG = (NCH - NB) // NB
TAIL = NCH - NB * (NG + 1)
NPAD = 10240
ROWS_PER_SUB = NPAD // NS
ZR = 8


def _segsum_sc(h, src, dst):
    mesh = plsc.VectorSubcoreMesh(core_axis_name="core", subcore_axis_name="subcore")

    @functools.partial(
        pl.kernel,
        out_type=[
            jax.ShapeDtypeStruct((NPAD, D), jnp.float32),
            jax.ShapeDtypeStruct((NPAD, D), jnp.float32),
        ],
        mesh=mesh,
        scratch_types=[
            pltpu.VMEM((C,), jnp.int32),
            pltpu.VMEM((C,), jnp.int32),
            pltpu.VMEM((C,), jnp.int32),
            pltpu.VMEM((C,), jnp.int32),
            pltpu.VMEM((C,), jnp.int32),
            pltpu.VMEM((C,), jnp.int32),
            pltpu.VMEM((C,), jnp.int32),
            pltpu.VMEM((C,), jnp.int32),
            pltpu.VMEM((C, D), jnp.float32),
            pltpu.VMEM((C, D), jnp.float32),
            pltpu.VMEM((C, D), jnp.float32),
            pltpu.VMEM((C, D), jnp.float32),
            pltpu.VMEM((ZR, D), jnp.float32),
            pltpu.VMEM_SHARED((NPAD, D), jnp.float32),
            pltpu.SemaphoreType.DMA,
            pltpu.SemaphoreType.DMA,
            pltpu.SemaphoreType.DMA,
            pltpu.SemaphoreType.DMA,
            pltpu.SemaphoreType.DMA,
            pltpu.SemaphoreType.DMA,
            pltpu.SemaphoreType.DMA,
            pltpu.SemaphoreType.DMA,
            pltpu.SemaphoreType.DMA,
            pltpu.SemaphoreType.DMA,
            pltpu.SemaphoreType.DMA,
            pltpu.SemaphoreType.DMA,
            pltpu.SemaphoreType.DMA,
            pltpu.SemaphoreType.DMA,
            pltpu.SemaphoreType.DMA,
            pltpu.SemaphoreType.DMA,
        ],
    )
    def seg_kernel(h_hbm, src_hbm, dst_hbm, o0_hbm, o1_hbm,
                   sv0, sv1, sv2, sv3, dv0, dv1, dv2, dv3,
                   r0, r1, r2, r3, zbuf, acc,
                   g0, g1, g2, g3, s0, s1, s2, s3,
                   ds0, ds1, ds2, ds3, is0, is1, is2, is3):
        srcv = [sv0, sv1, sv2, sv3]
        dstv = [dv0, dv1, dv2, dv3]
        rows = [r0, r1, r2, r3]
        gsem = [g0, g1, g2, g3]
        ssem = [s0, s1, s2, s3]
        dsem = [ds0, ds1, ds2, ds3]
        isem = [is0, is1, is2, is3]
        cid = lax.axis_index("core")
        sid = lax.axis_index("subcore")
        w = sid * NC + cid

        def start_src_idx(i, b):
            pltpu.async_copy(src_hbm.at[pl.ds(w * EP + i * C, C)], srcv[b],
                             isem[b])

        def start_dst_idx(i, b):
            pltpu.async_copy(dst_hbm.at[pl.ds(w * EP + i * C, C)], dstv[b],
                             dsem[b])

        def start_gather(i, b):
            pltpu.make_async_copy(src_hbm.at[pl.ds(w * EP + i * C, C)],
                                  srcv[b], isem[b]).wait()
            pltpu.async_copy(h_hbm.at[srcv[b]], rows[b], gsem[b])

        def finish_chunk(i, b):
            pltpu.make_async_copy(dst_hbm.at[pl.ds(w * EP + i * C, C)],
                                  dstv[b], dsem[b]).wait()
            pltpu.make_async_copy(h_hbm.at[srcv[b]], rows[b], gsem[b]).wait()
            pltpu.async_copy(rows[b], acc.at[dstv[b]], ssem[b], add=True)

        def wait_scatter(b):
            pltpu.make_async_copy(rows[b], acc.at[dstv[b]], ssem[b]).wait()

        for b in range(NB):
            start_src_idx(b, b)
            start_dst_idx(b, b)
        for b in range(NB):
            start_gather(b, b)

        @pl.loop(0, ZR)
        def _(i):
            @pl.loop(0, D, step=16)
            def _(j):
                zbuf[i, pl.ds(j, 16)] = jnp.zeros((16,), jnp.float32)

        @pl.loop(0, ROWS_PER_SUB // ZR)
        def _(k):
            pltpu.sync_copy(zbuf, acc.at[pl.ds(sid * ROWS_PER_SUB + k * ZR, ZR)])

        plsc.subcore_barrier()

        @pl.loop(0, NG)
        def _(g):
            i0 = g * NB
            for b in range(NB):
                finish_chunk(i0 + b, b)
                start_src_idx(i0 + NB + b, b)
            for b in range(NB):
                wait_scatter(b)
                start_dst_idx(i0 + NB + b, b)
                start_gather(i0 + NB + b, b)

        i0 = NG * NB
        for b in range(NB):
            finish_chunk(i0 + b, b)
            if b < TAIL:
                start_src_idx(i0 + NB + b, b)
        for t in range(TAIL):
            wait_scatter(t)
            start_dst_idx(i0 + NB + t, t)
            start_gather(i0 + NB + t, t)
        for t in range(TAIL):
            finish_chunk(i0 + NB + t, t)
        for b in range(NB):
            wait_scatter(b)

        plsc.subcore_barrier()

        row0 = sid * ROWS_PER_SUB

        @pl.when(cid == 0)
        def _():
            pltpu.sync_copy(acc.at[pl.ds(row0, ROWS_PER_SUB)],
                            o0_hbm.at[pl.ds(row0, ROWS_PER_SUB)])

        @pl.when(cid == 1)
        def _():
            pltpu.sync_copy(acc.at[pl.ds(row0, ROWS_PER_SUB)],
                            o1_hbm.at[pl.ds(row0, ROWS_PER_SUB)])

    return seg_kernel(h, src, dst)


ROW_BLK = 1000


def _mlp_body(final_relu, x_ref, p0_ref, p1_ref, wa_ref, ba_ref, wb_ref, bb_ref,
              o_ref):
    z = x_ref[...] + p0_ref[...] + p1_ref[...]
    t = jnp.dot(z, wa_ref[...], preferred_element_type=jnp.float32)
    t = jnp.maximum(t + ba_ref[...], 0.0)
    o = jnp.dot(t, wb_ref[...], preferred_element_type=jnp.float32)
    o = o + bb_ref[...]
    if final_relu:
        o = jnp.maximum(o, 0.0)
    o_ref[...] = o


def _mlp(x, p0, p1, Wa, ba, Wb, bb, final_relu):
    row_spec = pl.BlockSpec((ROW_BLK, D), lambda i: (i, 0))
    w_spec = pl.BlockSpec((D, D), lambda i: (0, 0))
    b_spec = pl.BlockSpec((1, D), lambda i: (0, 0))
    return pl.pallas_call(
        functools.partial(_mlp_body, final_relu),
        grid=(N // ROW_BLK,),
        in_specs=[row_spec, row_spec, row_spec, w_spec, b_spec, w_spec, b_spec],
        out_specs=row_spec,
        out_shape=jax.ShapeDtypeStruct((N, D), jnp.float32),
    )(x, p0, p1, Wa, ba.reshape(1, D), Wb, bb.reshape(1, D))


def kernel(x, edge_index, W1a, b1a, W1b, b1b, W2a, b2a, W2b, b2b):
    src = edge_index[0]
    dst = edge_index[1]
    p0, p1 = _segsum_sc(x, src, dst)
    h = _mlp(x, p0, p1, W1a, b1a, W1b, b1b, final_relu=True)
    q0, q1 = _segsum_sc(h, src, dst)
    return _mlp(h, q0, q1, W2a, b2a, W2b, b2b, final_relu=False)


# speedup vs baseline: 1.0397x; 1.0245x over previous
import functools

import jax
import jax.numpy as jnp
from jax import lax
from jax.experimental import pallas as pl
from jax.experimental.pallas import tpu as pltpu
from jax.experimental.pallas import tpu_sc as plsc

N = 10000
E = 320000
D = 128

NC = 2
NS = 16
NW = NC * NS
EP = E // NW
C = 80
NCH = EP // C
NB = 4
NG = (NCH - NB) // NB
TAIL = NCH - NB * (NG + 1)
NPAD = 10240
ROWS_PER_SUB = NPAD // NS
ZR = 8


def _segsum_sc(h, src, dst):
    mesh = plsc.VectorSubcoreMesh(core_axis_name="core", subcore_axis_name="subcore")

    @functools.partial(
        pl.kernel,
        out_type=[
            jax.ShapeDtypeStruct((NPAD, D), jnp.float32),
            jax.ShapeDtypeStruct((NPAD, D), jnp.float32),
        ],
        mesh=mesh,
        scratch_types=[
            pltpu.VMEM((C,), jnp.int32),
            pltpu.VMEM((C,), jnp.int32),
            pltpu.VMEM((C,), jnp.int32),
            pltpu.VMEM((C,), jnp.int32),
            pltpu.VMEM((C,), jnp.int32),
            pltpu.VMEM((C,), jnp.int32),
            pltpu.VMEM((C,), jnp.int32),
            pltpu.VMEM((C,), jnp.int32),
            pltpu.VMEM((C, D), jnp.float32),
            pltpu.VMEM((C, D), jnp.float32),
            pltpu.VMEM((C, D), jnp.float32),
            pltpu.VMEM((C, D), jnp.float32),
            pltpu.VMEM((ZR, D), jnp.float32),
            pltpu.VMEM_SHARED((NPAD, D), jnp.float32),
            pltpu.SemaphoreType.DMA,
            pltpu.SemaphoreType.DMA,
            pltpu.SemaphoreType.DMA,
            pltpu.SemaphoreType.DMA,
            pltpu.SemaphoreType.DMA,
            pltpu.SemaphoreType.DMA,
            pltpu.SemaphoreType.DMA,
            pltpu.SemaphoreType.DMA,
            pltpu.SemaphoreType.DMA,
            pltpu.SemaphoreType.DMA,
            pltpu.SemaphoreType.DMA,
            pltpu.SemaphoreType.DMA,
            pltpu.SemaphoreType.DMA,
            pltpu.SemaphoreType.DMA,
            pltpu.SemaphoreType.DMA,
            pltpu.SemaphoreType.DMA,
        ],
    )
    def seg_kernel(h_hbm, src_hbm, dst_hbm, o0_hbm, o1_hbm,
                   sv0, sv1, sv2, sv3, dv0, dv1, dv2, dv3,
                   r0, r1, r2, r3, zbuf, acc,
                   g0, g1, g2, g3, s0, s1, s2, s3,
                   ds0, ds1, ds2, ds3, is0, is1, is2, is3):
        srcv = [sv0, sv1, sv2, sv3]
        dstv = [dv0, dv1, dv2, dv3]
        rows = [r0, r1, r2, r3]
        gsem = [g0, g1, g2, g3]
        ssem = [s0, s1, s2, s3]
        dsem = [ds0, ds1, ds2, ds3]
        isem = [is0, is1, is2, is3]
        cid = lax.axis_index("core")
        sid = lax.axis_index("subcore")
        w = sid * NC + cid

        def start_src_idx(i, b):
            pltpu.async_copy(src_hbm.at[pl.ds(w * EP + i * C, C)], srcv[b],
                             isem[b])

        def start_dst_idx(i, b):
            pltpu.async_copy(dst_hbm.at[pl.ds(w * EP + i * C, C)], dstv[b],
                             dsem[b])

        def start_gather(i, b):
            pltpu.make_async_copy(src_hbm.at[pl.ds(w * EP + i * C, C)],
                                  srcv[b], isem[b]).wait()
            pltpu.async_copy(h_hbm.at[srcv[b]], rows[b], gsem[b])

        def finish_chunk(i, b):
            pltpu.make_async_copy(dst_hbm.at[pl.ds(w * EP + i * C, C)],
                                  dstv[b], dsem[b]).wait()
            pltpu.make_async_copy(h_hbm.at[srcv[b]], rows[b], gsem[b]).wait()
            pltpu.async_copy(rows[b], acc.at[dstv[b]], ssem[b], add=True)

        def wait_scatter(b):
            pltpu.make_async_copy(rows[b], acc.at[dstv[b]], ssem[b]).wait()

        for b in range(NB):
            start_src_idx(b, b)
            start_dst_idx(b, b)
        for b in range(NB):
            start_gather(b, b)

        @pl.loop(0, ZR)
        def _(i):
            @pl.loop(0, D, step=16)
            def _(j):
                zbuf[i, pl.ds(j, 16)] = jnp.zeros((16,), jnp.float32)

        @pl.loop(0, ROWS_PER_SUB // ZR)
        def _(k):
            pltpu.sync_copy(zbuf, acc.at[pl.ds(sid * ROWS_PER_SUB + k * ZR, ZR)])

        plsc.subcore_barrier()

        @pl.loop(0, NG)
        def _(g):
            i0 = g * NB
            for b in range(NB):
                finish_chunk(i0 + b, b)
                start_src_idx(i0 + NB + b, b)
            for b in range(NB):
                wait_scatter(b)
                start_dst_idx(i0 + NB + b, b)
                start_gather(i0 + NB + b, b)

        i0 = NG * NB
        for b in range(NB):
            finish_chunk(i0 + b, b)
            if b < TAIL:
                start_src_idx(i0 + NB + b, b)
        for t in range(TAIL):
            wait_scatter(t)
            start_dst_idx(i0 + NB + t, t)
            start_gather(i0 + NB + t, t)
        for t in range(TAIL):
            finish_chunk(i0 + NB + t, t)
        for b in range(NB):
            wait_scatter(b)

        plsc.subcore_barrier()

        row0 = sid * ROWS_PER_SUB

        @pl.when(cid == 0)
        def _():
            pltpu.sync_copy(acc.at[pl.ds(row0, ROWS_PER_SUB)],
                            o0_hbm.at[pl.ds(row0, ROWS_PER_SUB)])

        @pl.when(cid == 1)
        def _():
            pltpu.sync_copy(acc.at[pl.ds(row0, ROWS_PER_SUB)],
                            o1_hbm.at[pl.ds(row0, ROWS_PER_SUB)])

    return seg_kernel(h, src, dst)


ROW_BLK = 2000


def _mlp_body(final_relu, x_ref, p0_ref, p1_ref, wa_ref, ba_ref, wb_ref, bb_ref,
              o_ref):
    z = x_ref[...] + p0_ref[...] + p1_ref[...]
    t = jnp.dot(z, wa_ref[...], preferred_element_type=jnp.float32)
    t = jnp.maximum(t + ba_ref[...], 0.0)
    o = jnp.dot(t, wb_ref[...], preferred_element_type=jnp.float32)
    o = o + bb_ref[...]
    if final_relu:
        o = jnp.maximum(o, 0.0)
    o_ref[...] = o


def _mlp(x, p0, p1, Wa, ba, Wb, bb, final_relu):
    row_spec = pl.BlockSpec((ROW_BLK, D), lambda i: (i, 0))
    w_spec = pl.BlockSpec((D, D), lambda i: (0, 0))
    b_spec = pl.BlockSpec((1, D), lambda i: (0, 0))
    return pl.pallas_call(
        functools.partial(_mlp_body, final_relu),
        grid=(N // ROW_BLK,),
        in_specs=[row_spec, row_spec, row_spec, w_spec, b_spec, w_spec, b_spec],
        out_specs=row_spec,
        out_shape=jax.ShapeDtypeStruct((N, D), jnp.float32),
    )(x, p0, p1, Wa, ba.reshape(1, D), Wb, bb.reshape(1, D))


def kernel(x, edge_index, W1a, b1a, W1b, b1b, W2a, b2a, W2b, b2b):
    src = edge_index[0]
    dst = edge_index[1]
    p0, p1 = _segsum_sc(x, src, dst)
    h = _mlp(x, p0, p1, W1a, b1a, W1b, b1b, final_relu=True)
    q0, q1 = _segsum_sc(h, src, dst)
    return _mlp(h, q0, q1, W2a, b2a, W2b, b2b, final_relu=False)


# speedup vs baseline: 1.0502x; 1.0100x over previous
import functools

import jax
import jax.numpy as jnp
from jax import lax
from jax.experimental import pallas as pl
from jax.experimental.pallas import tpu as pltpu
from jax.experimental.pallas import tpu_sc as plsc

N = 10000
E = 320000
D = 128

NC = 2
NS = 16
NW = NC * NS
EP = E // NW
C = 80
NCH = EP // C
NB = 4
NG = (NCH - NB) // NB
TAIL = NCH - NB * (NG + 1)
NPAD = 10240
ROWS_PER_SUB = NPAD // NS
ZR = 8


def _segsum_sc(h, src, dst):
    mesh = plsc.VectorSubcoreMesh(core_axis_name="core", subcore_axis_name="subcore")

    @functools.partial(
        pl.kernel,
        out_type=[
            jax.ShapeDtypeStruct((NPAD, D), jnp.float32),
            jax.ShapeDtypeStruct((NPAD, D), jnp.float32),
        ],
        mesh=mesh,
        scratch_types=[
            pltpu.VMEM((C,), jnp.int32),
            pltpu.VMEM((C,), jnp.int32),
            pltpu.VMEM((C,), jnp.int32),
            pltpu.VMEM((C,), jnp.int32),
            pltpu.VMEM((C,), jnp.int32),
            pltpu.VMEM((C,), jnp.int32),
            pltpu.VMEM((C,), jnp.int32),
            pltpu.VMEM((C,), jnp.int32),
            pltpu.VMEM((C, D), jnp.float32),
            pltpu.VMEM((C, D), jnp.float32),
            pltpu.VMEM((C, D), jnp.float32),
            pltpu.VMEM((C, D), jnp.float32),
            pltpu.VMEM((ZR, D), jnp.float32),
            pltpu.VMEM_SHARED((NPAD, D), jnp.float32),
            pltpu.SemaphoreType.DMA,
            pltpu.SemaphoreType.DMA,
            pltpu.SemaphoreType.DMA,
            pltpu.SemaphoreType.DMA,
            pltpu.SemaphoreType.DMA,
            pltpu.SemaphoreType.DMA,
            pltpu.SemaphoreType.DMA,
            pltpu.SemaphoreType.DMA,
            pltpu.SemaphoreType.DMA,
            pltpu.SemaphoreType.DMA,
            pltpu.SemaphoreType.DMA,
            pltpu.SemaphoreType.DMA,
            pltpu.SemaphoreType.DMA,
            pltpu.SemaphoreType.DMA,
            pltpu.SemaphoreType.DMA,
            pltpu.SemaphoreType.DMA,
        ],
    )
    def seg_kernel(h_hbm, src_hbm, dst_hbm, o0_hbm, o1_hbm,
                   sv0, sv1, sv2, sv3, dv0, dv1, dv2, dv3,
                   r0, r1, r2, r3, zbuf, acc,
                   g0, g1, g2, g3, s0, s1, s2, s3,
                   ds0, ds1, ds2, ds3, is0, is1, is2, is3):
        srcv = [sv0, sv1, sv2, sv3]
        dstv = [dv0, dv1, dv2, dv3]
        rows = [r0, r1, r2, r3]
        gsem = [g0, g1, g2, g3]
        ssem = [s0, s1, s2, s3]
        dsem = [ds0, ds1, ds2, ds3]
        isem = [is0, is1, is2, is3]
        cid = lax.axis_index("core")
        sid = lax.axis_index("subcore")
        w = sid * NC + cid

        def start_src_idx(i, b):
            pltpu.async_copy(src_hbm.at[pl.ds(w * EP + i * C, C)], srcv[b],
                             isem[b])

        def start_dst_idx(i, b):
            pltpu.async_copy(dst_hbm.at[pl.ds(w * EP + i * C, C)], dstv[b],
                             dsem[b])

        def start_gather(i, b):
            pltpu.make_async_copy(src_hbm.at[pl.ds(w * EP + i * C, C)],
                                  srcv[b], isem[b]).wait()
            pltpu.async_copy(h_hbm.at[srcv[b]], rows[b], gsem[b])

        def finish_chunk(i, b):
            pltpu.make_async_copy(dst_hbm.at[pl.ds(w * EP + i * C, C)],
                                  dstv[b], dsem[b]).wait()
            pltpu.make_async_copy(h_hbm.at[srcv[b]], rows[b], gsem[b]).wait()
            pltpu.async_copy(rows[b], acc.at[dstv[b]], ssem[b], add=True)

        def wait_scatter(b):
            pltpu.make_async_copy(rows[b], acc.at[dstv[b]], ssem[b]).wait()

        for b in range(NB):
            start_src_idx(b, b)
            start_dst_idx(b, b)
        for b in range(NB):
            start_gather(b, b)

        @pl.loop(0, ZR)
        def _(i):
            @pl.loop(0, D, step=16)
            def _(j):
                zbuf[i, pl.ds(j, 16)] = jnp.zeros((16,), jnp.float32)

        @pl.loop(0, ROWS_PER_SUB // ZR)
        def _(k):
            pltpu.sync_copy(zbuf, acc.at[pl.ds(sid * ROWS_PER_SUB + k * ZR, ZR)])

        plsc.subcore_barrier()

        @pl.loop(0, NG)
        def _(g):
            i0 = g * NB
            for b in range(NB):
                finish_chunk(i0 + b, b)
                start_src_idx(i0 + NB + b, b)
            for b in range(NB):
                wait_scatter(b)
                start_dst_idx(i0 + NB + b, b)
                start_gather(i0 + NB + b, b)

        i0 = NG * NB
        for b in range(NB):
            finish_chunk(i0 + b, b)
            if b < TAIL:
                start_src_idx(i0 + NB + b, b)
        for t in range(TAIL):
            wait_scatter(t)
            start_dst_idx(i0 + NB + t, t)
            start_gather(i0 + NB + t, t)
        for t in range(TAIL):
            finish_chunk(i0 + NB + t, t)
        for b in range(NB):
            wait_scatter(b)

        plsc.subcore_barrier()

        row0 = sid * ROWS_PER_SUB

        @pl.when(cid == 0)
        def _():
            pltpu.sync_copy(acc.at[pl.ds(row0, ROWS_PER_SUB)],
                            o0_hbm.at[pl.ds(row0, ROWS_PER_SUB)])

        @pl.when(cid == 1)
        def _():
            pltpu.sync_copy(acc.at[pl.ds(row0, ROWS_PER_SUB)],
                            o1_hbm.at[pl.ds(row0, ROWS_PER_SUB)])

    return seg_kernel(h, src, dst)


ROW_BLK = 5000


def _mlp_body(final_relu, x_ref, p0_ref, p1_ref, wa_ref, ba_ref, wb_ref, bb_ref,
              o_ref):
    z = x_ref[...] + p0_ref[...] + p1_ref[...]
    t = jnp.dot(z, wa_ref[...], preferred_element_type=jnp.float32)
    t = jnp.maximum(t + ba_ref[...], 0.0)
    o = jnp.dot(t, wb_ref[...], preferred_element_type=jnp.float32)
    o = o + bb_ref[...]
    if final_relu:
        o = jnp.maximum(o, 0.0)
    o_ref[...] = o


def _mlp(x, p0, p1, Wa, ba, Wb, bb, final_relu):
    row_spec = pl.BlockSpec((ROW_BLK, D), lambda i: (i, 0))
    w_spec = pl.BlockSpec((D, D), lambda i: (0, 0))
    b_spec = pl.BlockSpec((1, D), lambda i: (0, 0))
    return pl.pallas_call(
        functools.partial(_mlp_body, final_relu),
        grid=(N // ROW_BLK,),
        in_specs=[row_spec, row_spec, row_spec, w_spec, b_spec, w_spec, b_spec],
        out_specs=row_spec,
        out_shape=jax.ShapeDtypeStruct((N, D), jnp.float32),
    )(x, p0, p1, Wa, ba.reshape(1, D), Wb, bb.reshape(1, D))


def kernel(x, edge_index, W1a, b1a, W1b, b1b, W2a, b2a, W2b, b2b):
    src = edge_index[0]
    dst = edge_index[1]
    p0, p1 = _segsum_sc(x, src, dst)
    h = _mlp(x, p0, p1, W1a, b1a, W1b, b1b, final_relu=True)
    q0, q1 = _segsum_sc(h, src, dst)
    return _mlp(h, q0, q1, W2a, b2a, W2b, b2b, final_relu=False)
